# baseline (speedup 1.0000x reference)
import functools

import jax
import jax.numpy as jnp
from jax import lax
from jax.experimental import pallas as pl
from jax.experimental.pallas import tpu as pltpu
from jax.experimental.pallas import tpu_sc as plsc

_S = 128
_D = 256
_TR = 128
_CH = 16 * _TR
_G = 49
_BINS = 16 * _G
_ACCR = 136
_BV = 128


def _encode(c):
    c = c.astype(jnp.int32)
    return ((c[:, 0] * _S + c[:, 1]) * _S + c[:, 2]) * _S + c[:, 3]


def _build_stream(src, pos, n):
    m_total = n + _BINS * 8 + _BV
    b = pos // _TR
    bidx = jnp.arange(_BINS, dtype=jnp.int32)
    lo = jnp.searchsorted(b, bidx, side="left").astype(jnp.int32)
    hi = jnp.searchsorted(b, bidx, side="right").astype(jnp.int32)
    cnt = hi - lo
    mpad = ((cnt + 7) // 8) * 8
    off = jnp.concatenate([jnp.zeros((1,), jnp.int32), jnp.cumsum(mpad)])[:_BINS]
    newidx = off[b] + (jnp.arange(n, dtype=jnp.int32) - lo[b])
    psrc = jnp.zeros((m_total,), jnp.int32).at[newidx].set(src)
    prel = jnp.zeros((m_total,), jnp.int32).at[newidx].set(pos - b * _TR)
    starts = off.reshape(_G, 16).T.astype(jnp.int32)
    counts = cnt.reshape(_G, 16).T.astype(jnp.int32)
    return psrc, prel, starts, counts


def _sc_body(in_feats, ot_feats, psrc_in, prel_in, psrc_ot, prel_ot,
             meta, zrows, out,
             acc_a, acc_b, zbuf, srcv, relv, rows, m_all, sem):
    s = lax.axis_index("c")
    t = lax.axis_index("s")

    @pl.when(t == 0)
    def _init_zeros():
        pltpu.sync_copy(zrows, zbuf)

    pltpu.sync_copy(meta.at[t], m_all)
    plsc.subcore_barrier()

    def run_stream(psrc, prel, feats, acc, start, v):
        nb = (v + _BV - 1) // _BV
        lane16 = lax.iota(jnp.int32, 16)

        def bbody(bi, carry):
            e = pl.multiple_of(start + bi * _BV, 8)
            pltpu.sync_copy(psrc.at[pl.ds(e, _BV)], srcv)
            pltpu.sync_copy(prel.at[pl.ds(e, _BV)], relv)
            for i in range(_BV // 16):
                l = bi * _BV + i * 16 + lane16
                valid = l < v
                sv = srcv[pl.ds(i * 16, 16)]
                rv = relv[pl.ds(i * 16, 16)]
                srcv[pl.ds(i * 16, 16)] = jnp.where(valid, sv, lane16)
                relv[pl.ds(i * 16, 16)] = jnp.where(
                    valid, rv, _TR + (lane16 & 7))
            pltpu.async_copy(feats.at[srcv], rows, sem).wait()
            ng = (jnp.minimum(v - bi * _BV, _BV) + 15) // 16

            def gbody(g, gcarry):
                goff = pl.multiple_of(g * 16, 16)
                relvec = relv[pl.ds(goff, 16)]
                for jj in range(16):
                    row_j = goff + jj
                    rel_j = relvec[jj]
                    for i in range(_D // 16):
                        vals = rows[row_j, pl.ds(16 * i, 16)]
                        plsc.addupdate(acc.at[rel_j, pl.ds(16 * i, 16)], vals)
                return gcarry

            lax.fori_loop(0, ng, gbody, 0)
            return carry

        lax.fori_loop(0, nb, bbody, 0)

    def chunk_body(r, carry):
        c = 2 * r + s

        @pl.when(c < _G)
        def _do_chunk():
            for z in range(4):
                pltpu.sync_copy(zbuf, acc_a.at[pl.ds(z * 32, 32)])
                pltpu.sync_copy(zbuf, acc_b.at[pl.ds(z * 32, 32)])
            pltpu.sync_copy(zbuf.at[pl.ds(0, 8)], acc_a.at[pl.ds(_TR, 8)])
            pltpu.sync_copy(zbuf.at[pl.ds(0, 8)], acc_b.at[pl.ds(_TR, 8)])
            mvec = m_all[pl.ds(pl.multiple_of(16 * c, 16), 16)]
            run_stream(psrc_in, prel_in, in_feats, acc_b, mvec[0], mvec[1])
            run_stream(psrc_ot, prel_ot, ot_feats, acc_a, mvec[2], mvec[3])
            row0 = pl.multiple_of(c * _CH + t * _TR, 8)
            full = jnp.logical_or(c < _G - 1, t < 13)

            @pl.when(full)
            def _write_full():
                pltpu.sync_copy(acc_a.at[pl.ds(0, _TR)],
                                out.at[pl.ds(row0, _TR), pl.ds(0, _D)])
                pltpu.sync_copy(acc_b.at[pl.ds(0, _TR)],
                                out.at[pl.ds(row0, _TR), pl.ds(_D, _D)])

            @pl.when(jnp.logical_and(c == _G - 1, t == 13))
            def _write_tail():
                pltpu.sync_copy(acc_a.at[pl.ds(0, 32)],
                                out.at[pl.ds(row0, 32), pl.ds(0, _D)])
                pltpu.sync_copy(acc_b.at[pl.ds(0, 32)],
                                out.at[pl.ds(row0, 32), pl.ds(_D, _D)])

        return carry

    lax.fori_loop(0, (_G + 1) // 2, chunk_body, 0)


def kernel(input_coords, input_feats, other_coords, other_feats):
    n1 = input_feats.shape[0]
    n2 = other_feats.shape[0]
    nt = n1 + n2
    k_in = _encode(input_coords)
    k_ot = _encode(other_coords)
    all_k = jnp.concatenate([k_in, k_ot])
    sk, sidx = lax.sort([all_k, jnp.arange(nt, dtype=jnp.int32)], num_keys=1)
    isnew = jnp.concatenate(
        [jnp.ones((1,), jnp.int32), (sk[1:] != sk[:-1]).astype(jnp.int32)])
    pos = jnp.cumsum(isnew) - 1
    is_in = sidx < n1
    idx_in = jnp.nonzero(is_in, size=n1)[0]
    idx_ot = jnp.nonzero(~is_in, size=n2)[0]
    psrc_in, prel_in, st_in, ct_in = _build_stream(sidx[idx_in], pos[idx_in], n1)
    psrc_ot, prel_ot, st_ot, ct_ot = _build_stream(
        sidx[idx_ot] - n1, pos[idx_ot], n2)
    meta = jnp.stack([st_in, ct_in, st_ot, ct_ot], axis=-1)
    meta = jnp.pad(meta, ((0, 0), (0, 0), (0, 12))).reshape(16, 16 * _G)
    zrows = jnp.zeros((32, _D), jnp.float32)

    run = functools.partial(
        pl.kernel,
        mesh=plsc.VectorSubcoreMesh(core_axis_name="c", subcore_axis_name="s"),
        out_type=jax.ShapeDtypeStruct((nt, 2 * _D), jnp.float32),
        scratch_types=[
            pltpu.VMEM((_ACCR, _D), jnp.float32),
            pltpu.VMEM((_ACCR, _D), jnp.float32),
            pltpu.VMEM_SHARED((32, _D), jnp.float32),
            pltpu.VMEM((_BV,), jnp.int32),
            pltpu.VMEM((_BV,), jnp.int32),
            pltpu.VMEM((_BV, _D), jnp.float32),
            pltpu.VMEM((16 * _G,), jnp.int32),
            pltpu.SemaphoreType.DMA,
        ],
    )(_sc_body)
    return run(input_feats, other_feats, psrc_in, prel_in, psrc_ot, prel_ot,
               meta, zrows)
